# baseline (speedup 1.0000x reference)
import functools

import jax
import jax.numpy as jnp
from jax.experimental import pallas as pl
from jax.experimental.pallas import tpu as pltpu

_SUB = 8
_CCHUNK = 256
_CGRP = _CCHUNK // _SUB


def _transh_kernel(
    ph_idx, pt_idx, nh_idx, nt_idx, pr_idx, nr_idx,
    ent_hbm,
    rel_ref,
    nrm_ref,
    out_ref,
    ent_vmem,
    prt, pwt, nrt, nwt,
    pha, pta, nha, nta,
    phb, ptb, nhb, ntb,
    copy_sem,
    *, margin, alpha, batch, dim, n_rows):
  n_groups = n_rows // _SUB
  n_cchunks = n_rows // _CCHUNK
  inv_dim = 1.0 / dim

  cp = pltpu.make_async_copy(ent_hbm, ent_vmem, copy_sem)
  cp.start()

  def rel_body(c, carry):
    base = c * _SUB
    for u in range(_SUB):
      gi = base + u
      pr = pr_idx[gi]
      nr = nr_idx[gi]
      prt[c, u] = rel_ref[pr, 0]
      pwt[c, u] = nrm_ref[pr, 0]
      nrt[c, u] = rel_ref[nr, 0]
      nwt[c, u] = nrm_ref[nr, 0]
    return carry
  jax.lax.fori_loop(0, n_groups, rel_body, 0)

  cp.wait()

  def gather_chunk(c, ht, tt, nh_t, nt_t):
    for g in range(_CGRP):
      base = (c * _CGRP + g) * _SUB
      for u in range(_SUB):
        gi = base + u
        ht[g, u] = ent_vmem[ph_idx[gi], 0]
        tt[g, u] = ent_vmem[pt_idx[gi], 0]
        nh_t[g, u] = ent_vmem[nh_idx[gi], 0]
        nt_t[g, u] = ent_vmem[nt_idx[gi], 0]

  ones_mat = jnp.ones((dim, dim), jnp.float32)

  def side(h, r, t, w):
    d = h - t
    dw = jnp.dot(d * w, ones_mat, preferred_element_type=jnp.float32)
    scores = (d + r) - dw * w
    dist = jnp.dot(jnp.abs(scores), ones_mat,
                   preferred_element_type=jnp.float32)
    q = jnp.abs(h) + jnp.abs(t) + (r * r) * inv_dim
    return dist, q

  def chunk_sums(c, ht, tt, nh_t, nt_t):
    sl = pl.ds(c * _CGRP, _CGRP)
    d2 = lambda x: x.reshape(_CCHUNK, dim)
    pd, p_q = side(d2(ht[...]), d2(prt[sl]), d2(tt[...]), d2(pwt[sl]))
    nd, n_q = side(d2(nh_t[...]), d2(nrt[sl]), d2(nt_t[...]), d2(nwt[sl]))
    rows = (c * _CCHUNK
            + jax.lax.broadcasted_iota(jnp.int32, (_CCHUNK, dim), 0))
    mask = (rows < batch).astype(jnp.float32)
    hinge = jnp.maximum(pd - nd + margin, 0.0)
    return jnp.sum(hinge * mask) * (1.0 / dim), jnp.sum((p_q + n_q) * mask)

  gather_chunk(0, pha, pta, nha, nta)

  def pipe_body(sc, carry):
    hinge_s, q_s = carry
    c0 = 2 * sc
    c1 = c0 + 1
    gather_chunk(c1, phb, ptb, nhb, ntb)
    h0, q0 = chunk_sums(c0, pha, pta, nha, nta)
    cnext = jnp.minimum(c0 + 2, n_cchunks - 1)
    gather_chunk(cnext, pha, pta, nha, nta)
    h1, q1 = chunk_sums(c1, phb, ptb, nhb, ntb)
    return (hinge_s + h0 + h1, q_s + q0 + q1)

  zero = jnp.float32(0.0)
  hinge_s, q_s = jax.lax.fori_loop(
      0, n_cchunks // 2, pipe_body, (zero, zero))

  inv_b = 1.0 / batch
  s = (hinge_s * inv_b + (alpha / 3.0) * (q_s * inv_b)
       - 4.0 * alpha / 3.0)
  out_ref[...] = jnp.reshape(s, (1, 1, 1))


def _transh_loss(ent_emb, rel_emb, norm_vec, pos_triplets, neg_triplets,
                 *, margin=4.0, alpha=0.01):
  B = int(pos_triplets.shape[0])
  E, D = int(ent_emb.shape[0]), int(ent_emb.shape[1])
  R = int(rel_emb.shape[0])

  n_rows = pl.cdiv(B, 2 * _CCHUNK) * 2 * _CCHUNK
  n_groups = n_rows // _SUB

  ent3 = ent_emb.astype(jnp.float32).reshape(E, 1, D)
  rel3 = rel_emb.astype(jnp.float32).reshape(R, 1, D)
  nrm3 = norm_vec.astype(jnp.float32).reshape(R, 1, D)

  def col(trip, j):
    c = trip[:, j].astype(jnp.int32)
    return jnp.pad(c, (0, n_rows - B))

  ph, pr, pt = col(pos_triplets, 0), col(pos_triplets, 1), col(pos_triplets, 2)
  nh, nr, nt = col(neg_triplets, 0), col(neg_triplets, 1), col(neg_triplets, 2)

  tiles_bytes = (n_rows * 4 + 8 * _CCHUNK) * D * 4
  vmem_bytes = (E * D + 2 * R * D) * 4 + tiles_bytes + (8 << 20)
  grid_spec = pltpu.PrefetchScalarGridSpec(
      num_scalar_prefetch=6,
      grid=(1,),
      in_specs=[pl.BlockSpec(memory_space=pl.ANY),
                pl.BlockSpec((R, 1, D), lambda c, *_: (0, 0, 0)),
                pl.BlockSpec((R, 1, D), lambda c, *_: (0, 0, 0))],
      out_specs=pl.BlockSpec((1, 1, 1), lambda c, *_: (0, 0, 0)),
      scratch_shapes=[pltpu.VMEM((E, 1, D), jnp.float32)]
                     + [pltpu.VMEM((n_groups, _SUB, D), jnp.float32)] * 4
                     + [pltpu.VMEM((_CGRP, _SUB, D), jnp.float32)] * 8
                     + [pltpu.SemaphoreType.DMA])
  out = pl.pallas_call(
      functools.partial(_transh_kernel, margin=float(margin),
                        alpha=float(alpha), batch=B, dim=D, n_rows=n_rows),
      out_shape=jax.ShapeDtypeStruct((1, 1, 1), jnp.float32),
      grid_spec=grid_spec,
      compiler_params=pltpu.CompilerParams(
          dimension_semantics=("arbitrary",),
          vmem_limit_bytes=int(min(58 * 2**20, vmem_bytes))),
      cost_estimate=pl.CostEstimate(
          flops=2 * n_rows * D * 30,
          transcendentals=0,
          bytes_accessed=(E * D + 2 * R * D + 4 * n_rows * D
                          + 6 * n_rows) * 4),
      name="transh_loss",
  )(ph, pt, nh, nt, pr, nr, ent3, rel3, nrm3)

  return out[0, 0, 0]


def kernel(ent_emb, rel_emb, norm_vec, pos_triplets, neg_triplets):
  return _transh_loss(ent_emb, rel_emb, norm_vec, pos_triplets, neg_triplets,
                      margin=4.0, alpha=0.01)


# speedup vs baseline: 1.0215x; 1.0215x over previous
import functools

import jax
import jax.numpy as jnp
from jax.experimental import pallas as pl
from jax.experimental.pallas import tpu as pltpu

_SUB = 8
_CCHUNK = 256
_CGRP = _CCHUNK // _SUB


def _transh_kernel(
    ph_idx, pt_idx, nh_idx, nt_idx, pr_idx, nr_idx,
    ent_hbm,
    rel_ref,
    nrm_ref,
    out_ref,
    ent_vmem,
    prt, pwt, nrt, nwt,
    pha, pta, nha, nta,
    phb, ptb, nhb, ntb,
    copy_sem,
    *, margin, alpha, batch, dim, n_rows):
  n_groups = n_rows // _SUB
  n_cchunks = n_rows // _CCHUNK
  inv_dim = 1.0 / dim

  cp = pltpu.make_async_copy(ent_hbm, ent_vmem, copy_sem)
  cp.start()

  def rel_body(c2, carry):
    for g in range(2):
      c = 2 * c2 + g
      base = c * _SUB
      for u in range(_SUB):
        gi = base + u
        pr = pr_idx[gi]
        nr = nr_idx[gi]
        prt[c, u] = rel_ref[pr, 0]
        pwt[c, u] = nrm_ref[pr, 0]
        nrt[c, u] = rel_ref[nr, 0]
        nwt[c, u] = nrm_ref[nr, 0]
    return carry
  jax.lax.fori_loop(0, n_groups // 2, rel_body, 0)

  cp.wait()

  def gather_chunk(c, ht, tt, nh_t, nt_t):
    for g in range(_CGRP):
      base = (c * _CGRP + g) * _SUB
      for u in range(_SUB):
        gi = base + u
        ht[g, u] = ent_vmem[ph_idx[gi], 0]
        tt[g, u] = ent_vmem[pt_idx[gi], 0]
        nh_t[g, u] = ent_vmem[nh_idx[gi], 0]
        nt_t[g, u] = ent_vmem[nt_idx[gi], 0]

  def side(h, r, t, w):
    d = h - t
    dw = jnp.sum(d * w, axis=2, keepdims=True)
    scores = (d + r) - dw * w
    dist = jnp.sum(jnp.abs(scores), axis=2, keepdims=True)
    q = jnp.sum(jnp.abs(h) + jnp.abs(t) + (r * r) * inv_dim,
                axis=2, keepdims=True)
    return dist, q

  def chunk_sums(c, ht, tt, nh_t, nt_t):
    sl = pl.ds(c * _CGRP, _CGRP)
    pd, p_q = side(ht[...], prt[sl], tt[...], pwt[sl])
    nd, n_q = side(nh_t[...], nrt[sl], nt_t[...], nwt[sl])
    rows = (c * _CCHUNK
            + _SUB * jax.lax.broadcasted_iota(jnp.int32, (_CGRP, _SUB, 1), 0)
            + jax.lax.broadcasted_iota(jnp.int32, (_CGRP, _SUB, 1), 1))
    mask = (rows < batch).astype(jnp.float32)
    hinge = jnp.maximum(pd - nd + margin, 0.0)
    return jnp.sum(hinge * mask), jnp.sum((p_q + n_q) * mask)

  gather_chunk(0, pha, pta, nha, nta)

  def pipe_body(sc, carry):
    hinge_s, q_s = carry
    c0 = 2 * sc
    c1 = c0 + 1
    gather_chunk(c1, phb, ptb, nhb, ntb)
    h0, q0 = chunk_sums(c0, pha, pta, nha, nta)
    cnext = jnp.minimum(c0 + 2, n_cchunks - 1)
    gather_chunk(cnext, pha, pta, nha, nta)
    h1, q1 = chunk_sums(c1, phb, ptb, nhb, ntb)
    return (hinge_s + h0 + h1, q_s + q0 + q1)

  zero = jnp.float32(0.0)
  hinge_s, q_s = jax.lax.fori_loop(
      0, n_cchunks // 2, pipe_body, (zero, zero))

  inv_b = 1.0 / batch
  s = (hinge_s * inv_b + (alpha / 3.0) * (q_s * inv_b)
       - 4.0 * alpha / 3.0)
  out_ref[...] = jnp.reshape(s, (1, 1, 1))


def _transh_loss(ent_emb, rel_emb, norm_vec, pos_triplets, neg_triplets,
                 *, margin=4.0, alpha=0.01):
  B = int(pos_triplets.shape[0])
  E, D = int(ent_emb.shape[0]), int(ent_emb.shape[1])
  R = int(rel_emb.shape[0])

  n_rows = pl.cdiv(B, 2 * _CCHUNK) * 2 * _CCHUNK
  n_groups = n_rows // _SUB

  ent3 = ent_emb.astype(jnp.float32).reshape(E, 1, D)
  rel3 = rel_emb.astype(jnp.float32).reshape(R, 1, D)
  nrm3 = norm_vec.astype(jnp.float32).reshape(R, 1, D)

  def col(trip, j):
    c = trip[:, j].astype(jnp.int32)
    return jnp.pad(c, (0, n_rows - B))

  ph, pr, pt = col(pos_triplets, 0), col(pos_triplets, 1), col(pos_triplets, 2)
  nh, nr, nt = col(neg_triplets, 0), col(neg_triplets, 1), col(neg_triplets, 2)

  tiles_bytes = (n_rows * 4 + 8 * _CCHUNK) * D * 4
  vmem_bytes = (E * D + 2 * R * D) * 4 + tiles_bytes + (8 << 20)
  grid_spec = pltpu.PrefetchScalarGridSpec(
      num_scalar_prefetch=6,
      grid=(1,),
      in_specs=[pl.BlockSpec(memory_space=pl.ANY),
                pl.BlockSpec((R, 1, D), lambda c, *_: (0, 0, 0)),
                pl.BlockSpec((R, 1, D), lambda c, *_: (0, 0, 0))],
      out_specs=pl.BlockSpec((1, 1, 1), lambda c, *_: (0, 0, 0)),
      scratch_shapes=[pltpu.VMEM((E, 1, D), jnp.float32)]
                     + [pltpu.VMEM((n_groups, _SUB, D), jnp.float32)] * 4
                     + [pltpu.VMEM((_CGRP, _SUB, D), jnp.float32)] * 8
                     + [pltpu.SemaphoreType.DMA])
  out = pl.pallas_call(
      functools.partial(_transh_kernel, margin=float(margin),
                        alpha=float(alpha), batch=B, dim=D, n_rows=n_rows),
      out_shape=jax.ShapeDtypeStruct((1, 1, 1), jnp.float32),
      grid_spec=grid_spec,
      compiler_params=pltpu.CompilerParams(
          dimension_semantics=("arbitrary",),
          vmem_limit_bytes=int(min(58 * 2**20, vmem_bytes))),
      cost_estimate=pl.CostEstimate(
          flops=2 * n_rows * D * 30,
          transcendentals=0,
          bytes_accessed=(E * D + 2 * R * D + 4 * n_rows * D
                          + 6 * n_rows) * 4),
      name="transh_loss",
  )(ph, pt, nh, nt, pr, nr, ent3, rel3, nrm3)

  return out[0, 0, 0]


def kernel(ent_emb, rel_emb, norm_vec, pos_triplets, neg_triplets):
  return _transh_loss(ent_emb, rel_emb, norm_vec, pos_triplets, neg_triplets,
                      margin=4.0, alpha=0.01)


# speedup vs baseline: 1.0258x; 1.0042x over previous
import functools

import jax
import jax.numpy as jnp
from jax.experimental import pallas as pl
from jax.experimental.pallas import tpu as pltpu

_SUB = 8
_CCHUNK = 256
_CGRP = _CCHUNK // _SUB


def _transh_kernel(
    ph_idx, pt_idx, nh_idx, nt_idx, pr_idx, nr_idx,
    ent_hbm,
    rel_ref,
    nrm_ref,
    out_ref,
    ent_vmem,
    prt, pwt, nrt, nwt,
    pha, pta, nha, nta,
    phb, ptb, nhb, ntb,
    copy_sem,
    *, margin, alpha, batch, dim, n_rows):
  n_groups = n_rows // _SUB
  n_cchunks = n_rows // _CCHUNK
  inv_dim = 1.0 / dim

  cp = pltpu.make_async_copy(ent_hbm, ent_vmem, copy_sem)
  cp.start()

  def rel_body(c, carry):
    base = c * _SUB
    for u in range(_SUB):
      gi = base + u
      pr = pr_idx[gi]
      nr = nr_idx[gi]
      prt[c, u] = rel_ref[pr, 0]
      pwt[c, u] = nrm_ref[pr, 0]
      nrt[c, u] = rel_ref[nr, 0]
      nwt[c, u] = nrm_ref[nr, 0]
    return carry
  jax.lax.fori_loop(0, n_groups, rel_body, 0)

  cp.wait()

  def gather_chunk(c, ht, tt, nh_t, nt_t):
    for g in range(_CGRP):
      base = (c * _CGRP + g) * _SUB
      for u in range(_SUB):
        gi = base + u
        ht[g, u] = ent_vmem[ph_idx[gi], 0]
        tt[g, u] = ent_vmem[pt_idx[gi], 0]
        nh_t[g, u] = ent_vmem[nh_idx[gi], 0]
        nt_t[g, u] = ent_vmem[nt_idx[gi], 0]

  def side(h, r, t, w):
    d = h - t
    dw = jnp.sum(d * w, axis=2, keepdims=True)
    scores = (d + r) - dw * w
    dist = jnp.sum(jnp.abs(scores), axis=2, keepdims=True)
    q = jnp.sum(jnp.abs(h) + jnp.abs(t) + (r * r) * inv_dim,
                axis=2, keepdims=True)
    return dist, q

  def chunk_sums(c, ht, tt, nh_t, nt_t):
    sl = pl.ds(c * _CGRP, _CGRP)
    pd, p_q = side(ht[...], prt[sl], tt[...], pwt[sl])
    nd, n_q = side(nh_t[...], nrt[sl], nt_t[...], nwt[sl])
    rows = (c * _CCHUNK
            + _SUB * jax.lax.broadcasted_iota(jnp.int32, (_CGRP, _SUB, 1), 0)
            + jax.lax.broadcasted_iota(jnp.int32, (_CGRP, _SUB, 1), 1))
    mask = (rows < batch).astype(jnp.float32)
    hinge = jnp.maximum(pd - nd + margin, 0.0)
    return jnp.sum(hinge * mask), jnp.sum((p_q + n_q) * mask)

  gather_chunk(0, pha, pta, nha, nta)

  def pipe_body(sc, carry):
    hinge_s, q_s = carry
    c0 = 2 * sc
    c1 = c0 + 1
    gather_chunk(c1, phb, ptb, nhb, ntb)
    h0, q0 = chunk_sums(c0, pha, pta, nha, nta)
    cnext = jnp.minimum(c0 + 2, n_cchunks - 1)
    gather_chunk(cnext, pha, pta, nha, nta)
    h1, q1 = chunk_sums(c1, phb, ptb, nhb, ntb)
    return (hinge_s + h0 + h1, q_s + q0 + q1)

  zero = jnp.float32(0.0)
  hinge_s, q_s = jax.lax.fori_loop(
      0, n_cchunks // 2, pipe_body, (zero, zero))

  inv_b = 1.0 / batch
  s = (hinge_s * inv_b + (alpha / 3.0) * (q_s * inv_b)
       - 4.0 * alpha / 3.0)
  out_ref[...] = jnp.reshape(s, (1, 1, 1))


def _transh_loss(ent_emb, rel_emb, norm_vec, pos_triplets, neg_triplets,
                 *, margin=4.0, alpha=0.01):
  B = int(pos_triplets.shape[0])
  E, D = int(ent_emb.shape[0]), int(ent_emb.shape[1])
  R = int(rel_emb.shape[0])

  n_rows = pl.cdiv(B, 2 * _CCHUNK) * 2 * _CCHUNK
  n_groups = n_rows // _SUB

  ent3 = ent_emb.astype(jnp.float32).reshape(E, 1, D)
  rel3 = rel_emb.astype(jnp.float32).reshape(R, 1, D)
  nrm3 = norm_vec.astype(jnp.float32).reshape(R, 1, D)

  def col(trip, j):
    c = trip[:, j].astype(jnp.int32)
    return jnp.pad(c, (0, n_rows - B))

  ph, pr, pt = col(pos_triplets, 0), col(pos_triplets, 1), col(pos_triplets, 2)
  nh, nr, nt = col(neg_triplets, 0), col(neg_triplets, 1), col(neg_triplets, 2)

  tiles_bytes = (n_rows * 4 + 8 * _CCHUNK) * D * 4
  vmem_bytes = (E * D + 2 * R * D) * 4 + tiles_bytes + (8 << 20)
  grid_spec = pltpu.PrefetchScalarGridSpec(
      num_scalar_prefetch=6,
      grid=(1,),
      in_specs=[pl.BlockSpec(memory_space=pl.ANY),
                pl.BlockSpec((R, 1, D), lambda c, *_: (0, 0, 0)),
                pl.BlockSpec((R, 1, D), lambda c, *_: (0, 0, 0))],
      out_specs=pl.BlockSpec((1, 1, 1), lambda c, *_: (0, 0, 0)),
      scratch_shapes=[pltpu.VMEM((E, 1, D), jnp.float32)]
                     + [pltpu.VMEM((n_groups, _SUB, D), jnp.float32)] * 4
                     + [pltpu.VMEM((_CGRP, _SUB, D), jnp.float32)] * 8
                     + [pltpu.SemaphoreType.DMA])
  out = pl.pallas_call(
      functools.partial(_transh_kernel, margin=float(margin),
                        alpha=float(alpha), batch=B, dim=D, n_rows=n_rows),
      out_shape=jax.ShapeDtypeStruct((1, 1, 1), jnp.float32),
      grid_spec=grid_spec,
      compiler_params=pltpu.CompilerParams(
          dimension_semantics=("arbitrary",),
          vmem_limit_bytes=int(min(58 * 2**20, vmem_bytes))),
      cost_estimate=pl.CostEstimate(
          flops=2 * n_rows * D * 30,
          transcendentals=0,
          bytes_accessed=(E * D + 2 * R * D + 4 * n_rows * D
                          + 6 * n_rows) * 4),
      name="transh_loss",
  )(ph, pt, nh, nt, pr, nr, ent3, rel3, nrm3)

  return out[0, 0, 0]


def kernel(ent_emb, rel_emb, norm_vec, pos_triplets, neg_triplets):
  return _transh_loss(ent_emb, rel_emb, norm_vec, pos_triplets, neg_triplets,
                      margin=4.0, alpha=0.01)


# speedup vs baseline: 1.0387x; 1.0126x over previous
import functools

import jax
import jax.numpy as jnp
from jax.experimental import pallas as pl
from jax.experimental.pallas import tpu as pltpu

_SUB = 8
_CCHUNK = 256
_CGRP = _CCHUNK // _SUB


def _transh_kernel(
    ph_idx, pt_idx, nh_idx, nt_idx, pr_idx, nr_idx,
    ent_hbm,
    rel_ref,
    nrm_ref,
    out_ref,
    ent_vmem,
    prt, pwt, nrt, nwt,
    pha, pta, nha, nta,
    phb, ptb, nhb, ntb,
    copy_sem,
    *, margin, alpha, batch, dim, n_rows):
  n_groups = n_rows // _SUB
  n_cchunks = n_rows // _CCHUNK
  inv_dim = 1.0 / dim

  cp = pltpu.make_async_copy(ent_hbm, ent_vmem, copy_sem)
  cp.start()

  def rel_body(c, carry):
    base = c * _SUB
    for u in range(_SUB):
      gi = base + u
      pr = pr_idx[gi]
      nr = nr_idx[gi]
      prt[c, u] = rel_ref[pr, 0]
      pwt[c, u] = nrm_ref[pr, 0]
      nrt[c, u] = rel_ref[nr, 0]
      nwt[c, u] = nrm_ref[nr, 0]
    return carry
  jax.lax.fori_loop(0, n_groups, rel_body, 0)

  cp.wait()

  def gather_chunk(c, ht, tt, nh_t, nt_t):
    for g in range(_CGRP):
      base = (c * _CGRP + g) * _SUB
      for u in range(_SUB):
        gi = base + u
        ht[g, u] = ent_vmem[ph_idx[gi], 0]
        tt[g, u] = ent_vmem[pt_idx[gi], 0]
        nh_t[g, u] = ent_vmem[nh_idx[gi], 0]
        nt_t[g, u] = ent_vmem[nt_idx[gi], 0]

  def side(h, r, t, w):
    d = h - t
    dw = jnp.sum(d * w, axis=2, keepdims=True)
    scores = (d + r) - dw * w
    dist = jnp.sum(jnp.abs(scores), axis=2, keepdims=True)
    q = jnp.sum(jnp.abs(h) + jnp.abs(t) + (r * r) * inv_dim,
                axis=2, keepdims=True)
    return dist, q

  _SUBBLK = 8

  def chunk_sums(c, ht, tt, nh_t, nt_t):
    hv = jnp.zeros((_SUBBLK, _SUB, 1), jnp.float32)
    qv = jnp.zeros((_SUBBLK, _SUB, 1), jnp.float32)
    for k in range(_CGRP // _SUBBLK):
      lsl = pl.ds(k * _SUBBLK, _SUBBLK)
      rsl = pl.ds(c * _CGRP + k * _SUBBLK, _SUBBLK)
      pd, p_q = side(ht[lsl], prt[rsl], tt[lsl], pwt[rsl])
      nd, n_q = side(nh_t[lsl], nrt[rsl], nt_t[lsl], nwt[rsl])
      rows = (c * _CCHUNK + k * _SUBBLK * _SUB
              + _SUB * jax.lax.broadcasted_iota(
                  jnp.int32, (_SUBBLK, _SUB, 1), 0)
              + jax.lax.broadcasted_iota(jnp.int32, (_SUBBLK, _SUB, 1), 1))
      mask = (rows < batch).astype(jnp.float32)
      hinge = jnp.maximum(pd - nd + margin, 0.0)
      hv = hv + hinge * mask
      qv = qv + (p_q + n_q) * mask
    return jnp.sum(hv), jnp.sum(qv)

  gather_chunk(0, pha, pta, nha, nta)

  def pipe_body(sc, carry):
    hinge_s, q_s = carry
    c0 = 2 * sc
    c1 = c0 + 1
    gather_chunk(c1, phb, ptb, nhb, ntb)
    h0, q0 = chunk_sums(c0, pha, pta, nha, nta)
    cnext = jnp.minimum(c0 + 2, n_cchunks - 1)
    gather_chunk(cnext, pha, pta, nha, nta)
    h1, q1 = chunk_sums(c1, phb, ptb, nhb, ntb)
    return (hinge_s + h0 + h1, q_s + q0 + q1)

  zero = jnp.float32(0.0)
  hinge_s, q_s = jax.lax.fori_loop(
      0, n_cchunks // 2, pipe_body, (zero, zero))

  inv_b = 1.0 / batch
  s = (hinge_s * inv_b + (alpha / 3.0) * (q_s * inv_b)
       - 4.0 * alpha / 3.0)
  out_ref[...] = jnp.reshape(s, (1, 1, 1))


def _transh_loss(ent_emb, rel_emb, norm_vec, pos_triplets, neg_triplets,
                 *, margin=4.0, alpha=0.01):
  B = int(pos_triplets.shape[0])
  E, D = int(ent_emb.shape[0]), int(ent_emb.shape[1])
  R = int(rel_emb.shape[0])

  n_rows = pl.cdiv(B, 2 * _CCHUNK) * 2 * _CCHUNK
  n_groups = n_rows // _SUB

  ent3 = ent_emb.astype(jnp.float32).reshape(E, 1, D)
  rel3 = rel_emb.astype(jnp.float32).reshape(R, 1, D)
  nrm3 = norm_vec.astype(jnp.float32).reshape(R, 1, D)

  def col(trip, j):
    c = trip[:, j].astype(jnp.int32)
    return jnp.pad(c, (0, n_rows - B))

  ph, pr, pt = col(pos_triplets, 0), col(pos_triplets, 1), col(pos_triplets, 2)
  nh, nr, nt = col(neg_triplets, 0), col(neg_triplets, 1), col(neg_triplets, 2)

  tiles_bytes = (n_rows * 4 + 8 * _CCHUNK) * D * 4
  vmem_bytes = (E * D + 2 * R * D) * 4 + tiles_bytes + (8 << 20)
  grid_spec = pltpu.PrefetchScalarGridSpec(
      num_scalar_prefetch=6,
      grid=(1,),
      in_specs=[pl.BlockSpec(memory_space=pl.ANY),
                pl.BlockSpec((R, 1, D), lambda c, *_: (0, 0, 0)),
                pl.BlockSpec((R, 1, D), lambda c, *_: (0, 0, 0))],
      out_specs=pl.BlockSpec((1, 1, 1), lambda c, *_: (0, 0, 0)),
      scratch_shapes=[pltpu.VMEM((E, 1, D), jnp.float32)]
                     + [pltpu.VMEM((n_groups, _SUB, D), jnp.float32)] * 4
                     + [pltpu.VMEM((_CGRP, _SUB, D), jnp.float32)] * 8
                     + [pltpu.SemaphoreType.DMA])
  out = pl.pallas_call(
      functools.partial(_transh_kernel, margin=float(margin),
                        alpha=float(alpha), batch=B, dim=D, n_rows=n_rows),
      out_shape=jax.ShapeDtypeStruct((1, 1, 1), jnp.float32),
      grid_spec=grid_spec,
      compiler_params=pltpu.CompilerParams(
          dimension_semantics=("arbitrary",),
          vmem_limit_bytes=int(min(58 * 2**20, vmem_bytes))),
      cost_estimate=pl.CostEstimate(
          flops=2 * n_rows * D * 30,
          transcendentals=0,
          bytes_accessed=(E * D + 2 * R * D + 4 * n_rows * D
                          + 6 * n_rows) * 4),
      name="transh_loss",
  )(ph, pt, nh, nt, pr, nr, ent3, rel3, nrm3)

  return out[0, 0, 0]


def kernel(ent_emb, rel_emb, norm_vec, pos_triplets, neg_triplets):
  return _transh_loss(ent_emb, rel_emb, norm_vec, pos_triplets, neg_triplets,
                      margin=4.0, alpha=0.01)
